# Initial kernel scaffold; baseline (speedup 1.0000x reference)
import jax
import jax.numpy as jnp
from jax.experimental import pallas as pl


def kernel(x0, pos0, x1):
    raise NotImplementedError("write your pallas kernel here")


# speedup vs baseline: 1.2052x; 1.2052x over previous
import functools

import jax
import jax.numpy as jnp
from jax import lax
from jax.experimental import pallas as pl
from jax.experimental.pallas import tpu as pltpu
from jax.experimental.pallas import tpu_sc as plsc

B, T0, T1, D = 8, 2048, 128, 512
NW = 32
PW = (B * T0) // NW
CH = 128
NCH = PW // CH
L = 16

_mesh = plsc.VectorSubcoreMesh(core_axis_name="c", subcore_axis_name="s")


@functools.partial(
    pl.kernel,
    out_type=jax.ShapeDtypeStruct((B * T0, 2 * D), jnp.float32),
    mesh=_mesh,
    scratch_types=[
        pltpu.VMEM((T1,), jnp.int32),
        pltpu.VMEM((NCH, CH), jnp.int32),
        pltpu.VMEM((CH, D), jnp.float32),
        pltpu.SemaphoreType.DMA,
    ],
    compiler_params=pltpu.CompilerParams(needs_layout_passes=False),
)
def _merge_sc(x0_hbm, pos_hbm, x1_hbm, out_hbm, pos_v, idx_v, buf, sem):
    cid = lax.axis_index("c")
    sid = lax.axis_index("s")
    wid = sid * 2 + cid
    base = wid * PW
    b = base // T0
    t0 = base % T0

    pltpu.sync_copy(pos_hbm.at[pl.ds(b * T1, T1)], pos_v)

    lanes = lax.iota(jnp.int32, L)
    for v in range(PW // L):
        t_vec = t0 + v * L + lanes
        j = jnp.zeros((L,), jnp.int32)
        for step in (64, 32, 16, 8, 4, 2, 1):
            cand = j + step
            vals = plsc.load_gather(pos_v, [cand])
            j = jnp.where(vals <= t_vec, cand, j)
        idx_v[v * L // CH, pl.ds((v * L) % CH, L)] = j + b * T1

    for ch in range(NCH):
        rbase = base + ch * CH
        pltpu.async_copy(x1_hbm.at[idx_v.at[ch]], buf, sem).wait()
        pltpu.sync_copy(buf, out_hbm.at[pl.ds(rbase, CH), pl.ds(D, D)])
        pltpu.sync_copy(x0_hbm.at[pl.ds(rbase, CH)], buf)
        pltpu.sync_copy(buf, out_hbm.at[pl.ds(rbase, CH), pl.ds(0, D)])


def kernel(x0, pos0, x1):
    x0f = jnp.reshape(x0, (B * T0, D))
    posf = jnp.reshape(pos0[:, :T1], (B * T1,))
    x1f = jnp.reshape(x1, (B * T1, D))
    out = _merge_sc(x0f, posf, x1f)
    return jnp.reshape(out, (B, T0, 2 * D))


# speedup vs baseline: 1.4343x; 1.1901x over previous
import functools

import jax
import jax.numpy as jnp
from jax import lax
from jax.experimental import pallas as pl
from jax.experimental.pallas import tpu as pltpu
from jax.experimental.pallas import tpu_sc as plsc

B, T0, T1, D = 8, 2048, 128, 512
NW = 32
PW = (B * T0) // NW
CH = 64
NCH = PW // CH
NSLOT = 3
L = 16

_JOBS = [("x", 0), ("x", 1), ("x", 2)]
for _c in range(3, NCH):
    _JOBS += [("y", _c - 3), ("x", _c)]
_JOBS += [("y", _c2) for _c2 in range(NCH - 3, NCH)]
assert len(_JOBS) == 2 * NCH

_mesh = plsc.VectorSubcoreMesh(core_axis_name="c", subcore_axis_name="s")


@functools.partial(
    pl.kernel,
    out_type=jax.ShapeDtypeStruct((B * T0, 2 * D), jnp.float32),
    mesh=_mesh,
    scratch_types=[
        pltpu.VMEM((T1,), jnp.int32),
        pltpu.VMEM((NCH, CH), jnp.int32),
        pltpu.VMEM((NSLOT, CH, D), jnp.float32),
    ] + [pltpu.SemaphoreType.DMA] * (2 * NSLOT),
    compiler_params=pltpu.CompilerParams(needs_layout_passes=False),
)
def _merge_sc(x0_hbm, pos_hbm, x1_hbm, out_hbm, pos_v, idx_v, buf,
              si0, si1, si2, so0, so1, so2):
    isem = (si0, si1, si2)
    osem = (so0, so1, so2)
    cid = lax.axis_index("c")
    sid = lax.axis_index("s")
    wid = sid * 2 + cid
    base = wid * PW
    b = base // T0
    t0 = base % T0

    def issue_in(j, s):
        kind, c = _JOBS[j]
        if kind == "x":
            return pltpu.async_copy(
                x0_hbm.at[pl.ds(base + c * CH, CH)], buf.at[s], isem[s])
        return pltpu.async_copy(x1_hbm.at[idx_v.at[c]], buf.at[s], isem[s])

    def issue_out(j, s):
        kind, c = _JOBS[j]
        col = 0 if kind == "x" else D
        return pltpu.async_copy(
            buf.at[s], out_hbm.at[pl.ds(base + c * CH, CH), pl.ds(col, D)],
            osem[s])

    pltpu.sync_copy(pos_hbm.at[pl.ds(b * T1, T1)], pos_v)

    in_h = [None] * len(_JOBS)
    out_h = [None] * len(_JOBS)
    for j in range(NSLOT):
        in_h[j] = issue_in(j, j)

    lanes = lax.iota(jnp.int32, L)
    for v in range(PW // L):
        t_vec = t0 + v * L + lanes
        j = jnp.zeros((L,), jnp.int32)
        for step in (64, 32, 16, 8, 4, 2, 1):
            cand = j + step
            vals = plsc.load_gather(pos_v, [cand])
            j = jnp.where(vals <= t_vec, cand, j)
        idx_v[v * L // CH, pl.ds((v * L) % CH, L)] = j + b * T1

    for j in range(len(_JOBS)):
        s = j % NSLOT
        in_h[j].wait()
        out_h[j] = issue_out(j, s)
        if j + NSLOT < len(_JOBS):
            out_h[j].wait()
            in_h[j + NSLOT] = issue_in(j + NSLOT, s)
    for j in range(len(_JOBS) - NSLOT, len(_JOBS)):
        out_h[j].wait()


def kernel(x0, pos0, x1):
    x0f = jnp.reshape(x0, (B * T0, D))
    posf = jnp.reshape(pos0[:, :T1], (B * T1,))
    x1f = jnp.reshape(x1, (B * T1, D))
    out = _merge_sc(x0f, posf, x1f)
    return jnp.reshape(out, (B, T0, 2 * D))


# speedup vs baseline: 1.5048x; 1.0491x over previous
import functools

import jax
import jax.numpy as jnp
from jax import lax
from jax.experimental import pallas as pl
from jax.experimental.pallas import tpu as pltpu
from jax.experimental.pallas import tpu_sc as plsc

B, T0, T1, D = 8, 2048, 128, 512
NW = 32
PW = (B * T0) // NW
CH = 32
NCH = PW // CH
NSLOT = 6
L = 16

_JOBS = [("x", 0), ("x", 1), ("x", 2)]
for _c in range(3, NCH):
    _JOBS += [("y", _c - 3), ("x", _c)]
_JOBS += [("y", _c2) for _c2 in range(NCH - 3, NCH)]
assert len(_JOBS) == 2 * NCH

_mesh = plsc.VectorSubcoreMesh(core_axis_name="c", subcore_axis_name="s")


@functools.partial(
    pl.kernel,
    out_type=jax.ShapeDtypeStruct((B * T0, 2 * D), jnp.float32),
    mesh=_mesh,
    scratch_types=[
        pltpu.VMEM((T1,), jnp.int32),
        pltpu.VMEM((NCH, CH), jnp.int32),
        pltpu.VMEM((NSLOT, CH, D), jnp.float32),
    ] + [pltpu.SemaphoreType.DMA] * (2 * NSLOT),
    compiler_params=pltpu.CompilerParams(needs_layout_passes=False),
)
def _merge_sc(x0_hbm, pos_hbm, x1_hbm, out_hbm, pos_v, idx_v, buf, *sems):
    isem = sems[:NSLOT]
    osem = sems[NSLOT:]
    cid = lax.axis_index("c")
    sid = lax.axis_index("s")
    wid = sid * 2 + cid
    base = wid * PW
    b = base // T0
    t0 = base % T0

    def issue_in(j, s):
        kind, c = _JOBS[j]
        if kind == "x":
            return pltpu.async_copy(
                x0_hbm.at[pl.ds(base + c * CH, CH)], buf.at[s], isem[s])
        return pltpu.async_copy(x1_hbm.at[idx_v.at[c]], buf.at[s], isem[s])

    def issue_out(j, s):
        kind, c = _JOBS[j]
        col = 0 if kind == "x" else D
        return pltpu.async_copy(
            buf.at[s], out_hbm.at[pl.ds(base + c * CH, CH), pl.ds(col, D)],
            osem[s])

    pltpu.sync_copy(pos_hbm.at[pl.ds(b * T1, T1)], pos_v)

    in_h = [None] * len(_JOBS)
    out_h = [None] * len(_JOBS)
    for j in range(NSLOT):
        in_h[j] = issue_in(j, j)

    lanes = lax.iota(jnp.int32, L)
    for v in range(PW // L):
        t_vec = t0 + v * L + lanes
        j = jnp.zeros((L,), jnp.int32)
        for step in (64, 32, 16, 8, 4, 2, 1):
            cand = j + step
            vals = plsc.load_gather(pos_v, [cand])
            j = jnp.where(vals <= t_vec, cand, j)
        idx_v[v * L // CH, pl.ds((v * L) % CH, L)] = j + b * T1

    for j in range(len(_JOBS)):
        s = j % NSLOT
        in_h[j].wait()
        out_h[j] = issue_out(j, s)
        if j + NSLOT < len(_JOBS):
            out_h[j].wait()
            in_h[j + NSLOT] = issue_in(j + NSLOT, s)
    for j in range(len(_JOBS) - NSLOT, len(_JOBS)):
        out_h[j].wait()


def kernel(x0, pos0, x1):
    x0f = jnp.reshape(x0, (B * T0, D))
    posf = jnp.reshape(pos0[:, :T1], (B * T1,))
    x1f = jnp.reshape(x1, (B * T1, D))
    out = _merge_sc(x0f, posf, x1f)
    return jnp.reshape(out, (B, T0, 2 * D))


# speedup vs baseline: 1.5542x; 1.0328x over previous
import functools

import jax
import jax.numpy as jnp
from jax import lax
from jax.experimental import pallas as pl
from jax.experimental.pallas import tpu as pltpu
from jax.experimental.pallas import tpu_sc as plsc

B, T0, T1, D = 8, 2048, 128, 512
NW = 32
PW = (B * T0) // NW
CH = 32
NCH = PW // CH
YS = 3
XS = 4
L = 16

_mesh = plsc.VectorSubcoreMesh(core_axis_name="c", subcore_axis_name="s")


@functools.partial(
    pl.kernel,
    out_type=jax.ShapeDtypeStruct((B * T0, 2 * D), jnp.float32),
    mesh=_mesh,
    scratch_types=[
        pltpu.VMEM((T1,), jnp.int32),
        pltpu.VMEM((NCH, CH), jnp.int32),
        pltpu.VMEM((YS, CH, D), jnp.float32),
        pltpu.VMEM_SHARED((16, XS, CH, D), jnp.float32),
    ] + [pltpu.SemaphoreType.DMA] * (2 * YS + 2 * XS),
    compiler_params=pltpu.CompilerParams(needs_layout_passes=False),
)
def _merge_sc(x0_hbm, pos_hbm, x1_hbm, out_hbm, pos_v, idx_v, ybuf, xbuf,
              *sems):
    yis = sems[:YS]
    yos = sems[YS:2 * YS]
    xis = sems[2 * YS:2 * YS + XS]
    xos = sems[2 * YS + XS:]
    cid = lax.axis_index("c")
    sid = lax.axis_index("s")
    wid = sid * 2 + cid
    base = wid * PW
    b = base // T0
    t0 = base % T0

    def x_in(c, s):
        return pltpu.async_copy(
            x0_hbm.at[pl.ds(base + c * CH, CH)], xbuf.at[sid, s], xis[s])

    def x_out(c, s):
        return pltpu.async_copy(
            xbuf.at[sid, s],
            out_hbm.at[pl.ds(base + c * CH, CH), pl.ds(0, D)], xos[s])

    def y_in(c, s):
        return pltpu.async_copy(x1_hbm.at[idx_v.at[c]], ybuf.at[s], yis[s])

    def y_out(c, s):
        return pltpu.async_copy(
            ybuf.at[s],
            out_hbm.at[pl.ds(base + c * CH, CH), pl.ds(D, D)], yos[s])

    xh_in = [None] * NCH
    xh_out = [None] * NCH
    for c in range(XS):
        xh_in[c] = x_in(c, c)
    pltpu.sync_copy(pos_hbm.at[pl.ds(b * T1, T1)], pos_v)

    lanes = lax.iota(jnp.int32, L)
    for v in range(PW // L):
        t_vec = t0 + v * L + lanes
        j = jnp.zeros((L,), jnp.int32)
        for step in (64, 32, 16, 8, 4, 2, 1):
            cand = j + step
            vals = plsc.load_gather(pos_v, [cand])
            j = jnp.where(vals <= t_vec, cand, j)
        idx_v[v * L // CH, pl.ds((v * L) % CH, L)] = j + b * T1

    yh_in = [None] * NCH
    yh_out = [None] * NCH
    for c in range(YS):
        yh_in[c] = y_in(c, c)

    for j in range(NCH):
        s = j % XS
        xh_in[j].wait()
        xh_out[j] = x_out(j, s)
        if j + XS < NCH:
            xh_out[j].wait()
            xh_in[j + XS] = x_in(j + XS, s)
        s = j % YS
        yh_in[j].wait()
        yh_out[j] = y_out(j, s)
        if j + YS < NCH:
            yh_out[j].wait()
            yh_in[j + YS] = y_in(j + YS, s)

    for j in range(NCH - XS, NCH):
        xh_out[j].wait()
    for j in range(NCH - YS, NCH):
        yh_out[j].wait()


def kernel(x0, pos0, x1):
    x0f = jnp.reshape(x0, (B * T0, D))
    posf = jnp.reshape(pos0[:, :T1], (B * T1,))
    x1f = jnp.reshape(x1, (B * T1, D))
    out = _merge_sc(x0f, posf, x1f)
    return jnp.reshape(out, (B, T0, 2 * D))
